# baseline (speedup 1.0000x reference)
import functools

import jax
import jax.numpy as jnp
from jax import lax
from jax.experimental import pallas as pl
from jax.experimental.pallas import tpu as pltpu
from jax.experimental.pallas import tpu_sc as plsc

N = 10000
E = 320000
D = 128
NPAD = 10240
NC = 2
NS = 16
EPW = E // NS


def _sc_mesh():
    return plsc.VectorSubcoreMesh(
        core_axis_name="c", subcore_axis_name="s", num_cores=NC, num_subcores=NS
    )


@functools.partial(
    pl.kernel,
    out_type=jax.ShapeDtypeStruct((NC, NPAD), jnp.float32),
    mesh=_sc_mesh(),
    scratch_types=[
        pltpu.VMEM((EPW,), jnp.int32),
        pltpu.VMEM((EPW,), jnp.int32),
        pltpu.VMEM((EPW,), jnp.float32),
        pltpu.VMEM((EPW,), jnp.float32),
        pltpu.VMEM_SHARED((NPAD,), jnp.float32),
        pltpu.VMEM_SHARED((NPAD,), jnp.float32),
    ],
)
def _sc_segment_weights(
    dstp_hbm, srcp_hbm, dstn_hbm, srcn_hbm, ones_hbm, zeros_hbm,
    out_hbm,
    dst_v, src_v, ones_v, w_v,
    cnt_sh, s_sh,
):
    c = lax.axis_index("c")
    s = lax.axis_index("s")

    @pl.when(c == 0)
    def _():
        pltpu.sync_copy(dstp_hbm.at[s], dst_v)
        pltpu.sync_copy(srcp_hbm.at[s], src_v)

    @pl.when(c == 1)
    def _():
        pltpu.sync_copy(dstn_hbm.at[s], dst_v)
        pltpu.sync_copy(srcn_hbm.at[s], src_v)

    pltpu.sync_copy(ones_hbm, ones_v)

    @pl.when(s == 0)
    def _():
        pltpu.sync_copy(zeros_hbm, cnt_sh)
        pltpu.sync_copy(zeros_hbm, s_sh)

    plsc.subcore_barrier()

    pltpu.sync_copy(ones_v, cnt_sh.at[dst_v], add=True)

    plsc.subcore_barrier()

    pltpu.sync_copy(cnt_sh.at[dst_v], w_v)

    def _recip(i, carry):
        for t in range(10):
            sl = pl.ds(i * 160 + t * 16, 16)
            w_v[sl] = 1.0 / w_v[sl]
        return carry

    lax.fori_loop(0, EPW // 160, _recip, 0)
    pltpu.sync_copy(w_v, s_sh.at[src_v], add=True)

    plsc.subcore_barrier()

    @pl.when(s == 0)
    def _():
        pltpu.sync_copy(s_sh, out_hbm.at[c])


BN = 2000
NSTEPS = N // BN


def _tc_body(vt_ref, x_ref, wlp_ref, wln_ref, wrp_ref, wrn_ref,
             blp_ref, bln_ref, out_ref, acc_ref):
    k = pl.program_id(0)

    @pl.when(k == 0)
    def _():
        acc_ref[...] = jnp.zeros_like(acc_ref)

    acc_ref[...] += lax.dot_general(
        vt_ref[...], x_ref[...], (((0,), (0,)), ((), ())),
        preferred_element_type=jnp.float32,
    )

    @pl.when(k == NSTEPS - 1)
    def _():
        y = acc_ref[...]
        sp = y[0:1]
        sn = y[1:2]
        xs = y[2:3]
        r = (
            jnp.dot(sp, wlp_ref[...], preferred_element_type=jnp.float32)
            + jnp.dot(sn, wln_ref[...], preferred_element_type=jnp.float32)
            + jnp.dot(xs, wrp_ref[...], preferred_element_type=jnp.float32)
            + jnp.dot(xs, wrn_ref[...], preferred_element_type=jnp.float32)
        ) * (1.0 / N) + blp_ref[...] + bln_ref[...]
        out_ref[...] = jnp.broadcast_to(r, (8, 128))


_tc_combine = pl.pallas_call(
    _tc_body,
    grid=(NSTEPS,),
    in_specs=[
        pl.BlockSpec((BN, 8), lambda k: (k, 0)),
        pl.BlockSpec((BN, D), lambda k: (k, 0)),
        pl.BlockSpec((D, D), lambda k: (0, 0)),
        pl.BlockSpec((D, D), lambda k: (0, 0)),
        pl.BlockSpec((D, D), lambda k: (0, 0)),
        pl.BlockSpec((D, D), lambda k: (0, 0)),
        pl.BlockSpec((1, D), lambda k: (0, 0)),
        pl.BlockSpec((1, D), lambda k: (0, 0)),
    ],
    out_specs=pl.BlockSpec((8, D), lambda k: (0, 0)),
    out_shape=jax.ShapeDtypeStruct((8, D), jnp.float32),
    scratch_shapes=[pltpu.VMEM((8, D), jnp.float32)],
)


@jax.jit
def kernel(x, edge_index_pos, edge_index_neg,
           Wl_pos, bl_pos, Wr_pos, Wl_neg, bl_neg, Wr_neg):
    dstp = edge_index_pos[1].astype(jnp.int32).reshape(NS, EPW)
    srcp = edge_index_pos[0].astype(jnp.int32).reshape(NS, EPW)
    dstn = edge_index_neg[1].astype(jnp.int32).reshape(NS, EPW)
    srcn = edge_index_neg[0].astype(jnp.int32).reshape(NS, EPW)
    ones_vals = jnp.ones((EPW,), jnp.float32)
    zeros_np = jnp.zeros((NPAD,), jnp.float32)

    s2 = _sc_segment_weights(dstp, srcp, dstn, srcn, ones_vals, zeros_np)

    vt = jnp.concatenate(
        [s2[:, :N].T, jnp.ones((N, 1), jnp.float32), jnp.zeros((N, 5), jnp.float32)],
        axis=1,
    )
    out8 = _tc_combine(
        vt, x, Wl_pos.T, Wl_neg.T, Wr_pos.T, Wr_neg.T,
        bl_pos.reshape(1, D), bl_neg.reshape(1, D),
    )
    return out8[0:1]


# speedup vs baseline: 1.0004x; 1.0004x over previous
import functools

import jax
import jax.numpy as jnp
from jax import lax
from jax.experimental import pallas as pl
from jax.experimental.pallas import tpu as pltpu
from jax.experimental.pallas import tpu_sc as plsc

N = 10000
E = 320000
D = 128
NPAD = 10240
NC = 2
NS = 16
EPW = E // NS


PAIR_ROWS = 2 * E * 2 // 128
BR = 1000
DSTEPS = PAIR_ROWS // BR


def _deint_body(p_ref, n_ref, sel_ref, po_ref, no_ref):
    sel = sel_ref[...]
    po_ref[...] = lax.dot_general(
        p_ref[...].astype(jnp.float32), sel, (((1,), (0,)), ((), ())),
        preferred_element_type=jnp.float32,
    ).astype(jnp.int32)
    no_ref[...] = lax.dot_general(
        n_ref[...].astype(jnp.float32), sel, (((1,), (0,)), ((), ())),
        preferred_element_type=jnp.float32,
    ).astype(jnp.int32)


_tc_deint = pl.pallas_call(
    _deint_body,
    grid=(DSTEPS,),
    in_specs=[
        pl.BlockSpec((BR, 128), lambda k: (k, 0)),
        pl.BlockSpec((BR, 128), lambda k: (k, 0)),
        pl.BlockSpec((128, 64), lambda k: (0, 0)),
    ],
    out_specs=[
        pl.BlockSpec((BR, 64), lambda k: (k, 0)),
        pl.BlockSpec((BR, 64), lambda k: (k, 0)),
    ],
    out_shape=[
        jax.ShapeDtypeStruct((PAIR_ROWS, 64), jnp.int32),
        jax.ShapeDtypeStruct((PAIR_ROWS, 64), jnp.int32),
    ],
)


def _sc_mesh():
    return plsc.VectorSubcoreMesh(
        core_axis_name="c", subcore_axis_name="s", num_cores=NC, num_subcores=NS
    )


@functools.partial(
    pl.kernel,
    out_type=jax.ShapeDtypeStruct((NC, NPAD), jnp.float32),
    mesh=_sc_mesh(),
    scratch_types=[
        pltpu.VMEM((EPW,), jnp.int32),
        pltpu.VMEM((EPW,), jnp.int32),
        pltpu.VMEM((EPW,), jnp.float32),
        pltpu.VMEM((EPW,), jnp.float32),
        pltpu.VMEM_SHARED((NPAD,), jnp.float32),
        pltpu.VMEM_SHARED((NPAD,), jnp.float32),
    ],
)
def _sc_segment_weights(
    dstp_hbm, srcp_hbm, dstn_hbm, srcn_hbm, ones_hbm, zeros_hbm,
    out_hbm,
    dst_v, src_v, ones_v, w_v,
    cnt_sh, s_sh,
):
    c = lax.axis_index("c")
    s = lax.axis_index("s")

    @pl.when(c == 0)
    def _():
        pltpu.sync_copy(dstp_hbm.at[s], dst_v)
        pltpu.sync_copy(srcp_hbm.at[s], src_v)

    @pl.when(c == 1)
    def _():
        pltpu.sync_copy(dstn_hbm.at[s], dst_v)
        pltpu.sync_copy(srcn_hbm.at[s], src_v)

    pltpu.sync_copy(ones_hbm, ones_v)

    @pl.when(s == 0)
    def _():
        pltpu.sync_copy(zeros_hbm, cnt_sh)
        pltpu.sync_copy(zeros_hbm, s_sh)

    plsc.subcore_barrier()

    pltpu.sync_copy(ones_v, cnt_sh.at[dst_v], add=True)

    plsc.subcore_barrier()

    pltpu.sync_copy(cnt_sh.at[dst_v], w_v)

    def _recip(i, carry):
        for t in range(10):
            sl = pl.ds(i * 160 + t * 16, 16)
            w_v[sl] = 1.0 / w_v[sl]
        return carry

    lax.fori_loop(0, EPW // 160, _recip, 0)
    pltpu.sync_copy(w_v, s_sh.at[src_v], add=True)

    plsc.subcore_barrier()

    @pl.when(s == 0)
    def _():
        pltpu.sync_copy(s_sh, out_hbm.at[c])


BN = 2000
NSTEPS = N // BN


def _tc_body(vt_ref, x_ref, wlp_ref, wln_ref, wrp_ref, wrn_ref,
             blp_ref, bln_ref, out_ref, acc_ref):
    k = pl.program_id(0)

    @pl.when(k == 0)
    def _():
        acc_ref[...] = jnp.zeros_like(acc_ref)

    acc_ref[...] += lax.dot_general(
        vt_ref[...], x_ref[...], (((0,), (0,)), ((), ())),
        preferred_element_type=jnp.float32,
    )

    @pl.when(k == NSTEPS - 1)
    def _():
        y = acc_ref[...]
        sp = y[0:1]
        sn = y[1:2]
        xs = y[2:3]
        r = (
            jnp.dot(sp, wlp_ref[...], preferred_element_type=jnp.float32)
            + jnp.dot(sn, wln_ref[...], preferred_element_type=jnp.float32)
            + jnp.dot(xs, wrp_ref[...], preferred_element_type=jnp.float32)
            + jnp.dot(xs, wrn_ref[...], preferred_element_type=jnp.float32)
        ) * (1.0 / N) + blp_ref[...] + bln_ref[...]
        out_ref[...] = jnp.broadcast_to(r, (8, 128))


_tc_combine = pl.pallas_call(
    _tc_body,
    grid=(NSTEPS,),
    in_specs=[
        pl.BlockSpec((BN, 8), lambda k: (k, 0)),
        pl.BlockSpec((BN, D), lambda k: (k, 0)),
        pl.BlockSpec((D, D), lambda k: (0, 0)),
        pl.BlockSpec((D, D), lambda k: (0, 0)),
        pl.BlockSpec((D, D), lambda k: (0, 0)),
        pl.BlockSpec((D, D), lambda k: (0, 0)),
        pl.BlockSpec((1, D), lambda k: (0, 0)),
        pl.BlockSpec((1, D), lambda k: (0, 0)),
    ],
    out_specs=pl.BlockSpec((8, D), lambda k: (0, 0)),
    out_shape=jax.ShapeDtypeStruct((8, D), jnp.float32),
    scratch_shapes=[pltpu.VMEM((8, D), jnp.float32)],
)


@jax.jit
def kernel(x, edge_index_pos, edge_index_neg,
           Wl_pos, bl_pos, Wr_pos, Wl_neg, bl_neg, Wr_neg):
    if edge_index_pos.dtype == jnp.int32:
        srcp = edge_index_pos[0].reshape(NS, EPW)
        dstp = edge_index_pos[1].reshape(NS, EPW)
        srcn = edge_index_neg[0].reshape(NS, EPW)
        dstn = edge_index_neg[1].reshape(NS, EPW)
    else:
        pairs_p = lax.bitcast_convert_type(
            edge_index_pos, jnp.int32).reshape(PAIR_ROWS, 128)
        pairs_n = lax.bitcast_convert_type(
            edge_index_neg, jnp.int32).reshape(PAIR_ROWS, 128)
        sel = (jnp.arange(128)[:, None] == 2 * jnp.arange(64)[None, :]).astype(
            jnp.float32)
        pc, nc = _tc_deint(pairs_p, pairs_n, sel)
        pc = pc.reshape(2, NS, EPW)
        nc = nc.reshape(2, NS, EPW)
        srcp, dstp = pc[0], pc[1]
        srcn, dstn = nc[0], nc[1]

    ones_vals = jnp.ones((EPW,), jnp.float32)
    zeros_np = jnp.zeros((NPAD,), jnp.float32)

    s2 = _sc_segment_weights(dstp, srcp, dstn, srcn, ones_vals, zeros_np)

    vt = jnp.concatenate(
        [s2[:, :N].T, jnp.ones((N, 1), jnp.float32), jnp.zeros((N, 5), jnp.float32)],
        axis=1,
    )
    out8 = _tc_combine(
        vt, x, Wl_pos.T, Wl_neg.T, Wr_pos.T, Wr_neg.T,
        bl_pos.reshape(1, D), bl_neg.reshape(1, D),
    )
    return out8[0:1]


# speedup vs baseline: 1.2051x; 1.2047x over previous
import functools

import jax
import jax.numpy as jnp
from jax import lax
from jax.experimental import pallas as pl
from jax.experimental.pallas import tpu as pltpu
from jax.experimental.pallas import tpu_sc as plsc

N = 10000
E = 320000
D = 128
NPAD = 10240
NC = 2
NS = 16
EPW = E // NS


def _sc_mesh():
    return plsc.VectorSubcoreMesh(
        core_axis_name="c", subcore_axis_name="s", num_cores=NC, num_subcores=NS
    )


@functools.partial(
    pl.kernel,
    out_type=jax.ShapeDtypeStruct((NC, NPAD), jnp.float32),
    mesh=_sc_mesh(),
    scratch_types=[
        pltpu.VMEM((EPW,), jnp.int32),
        pltpu.VMEM((EPW,), jnp.int32),
        pltpu.VMEM((EPW,), jnp.float32),
        pltpu.VMEM((EPW,), jnp.float32),
        pltpu.VMEM_SHARED((NPAD,), jnp.float32),
        pltpu.VMEM_SHARED((NPAD,), jnp.float32),
    ],
)
def _sc_segment_weights(
    dstp_hbm, srcp_hbm, dstn_hbm, srcn_hbm, ones_hbm, zeros_hbm,
    out_hbm,
    dst_v, src_v, ones_v, w_v,
    cnt_sh, s_sh,
):
    c = lax.axis_index("c")
    s = lax.axis_index("s")

    base = s * EPW

    @pl.when(c == 0)
    def _():
        pltpu.sync_copy(dstp_hbm.at[pl.ds(base, EPW)], dst_v)
        pltpu.sync_copy(srcp_hbm.at[pl.ds(base, EPW)], src_v)

    @pl.when(c == 1)
    def _():
        pltpu.sync_copy(dstn_hbm.at[pl.ds(base, EPW)], dst_v)
        pltpu.sync_copy(srcn_hbm.at[pl.ds(base, EPW)], src_v)

    pltpu.sync_copy(ones_hbm, ones_v)

    @pl.when(s == 0)
    def _():
        pltpu.sync_copy(zeros_hbm, cnt_sh)
        pltpu.sync_copy(zeros_hbm, s_sh)

    plsc.subcore_barrier()

    pltpu.sync_copy(ones_v, cnt_sh.at[dst_v], add=True)

    plsc.subcore_barrier()

    pltpu.sync_copy(cnt_sh.at[dst_v], w_v)

    def _recip(i, carry):
        for t in range(10):
            sl = pl.ds(i * 160 + t * 16, 16)
            w_v[sl] = 1.0 / w_v[sl]
        return carry

    lax.fori_loop(0, EPW // 160, _recip, 0)
    pltpu.sync_copy(w_v, s_sh.at[src_v], add=True)

    plsc.subcore_barrier()

    @pl.when(s == 0)
    def _():
        pltpu.sync_copy(s_sh, out_hbm.at[c])


BN = 2000
NSTEPS = N // BN


def _tc_body(vt_ref, x_ref, wlp_ref, wln_ref, wrp_ref, wrn_ref,
             blp_ref, bln_ref, out_ref, acc_ref):
    k = pl.program_id(0)

    @pl.when(k == 0)
    def _():
        acc_ref[...] = jnp.zeros_like(acc_ref)

    acc_ref[...] += lax.dot_general(
        vt_ref[...], x_ref[...], (((0,), (0,)), ((), ())),
        preferred_element_type=jnp.float32,
    )

    @pl.when(k == NSTEPS - 1)
    def _():
        y = acc_ref[...]
        sp = y[0:1]
        sn = y[1:2]
        xs = y[2:3]
        r = (
            jnp.dot(sp, wlp_ref[...], preferred_element_type=jnp.float32)
            + jnp.dot(sn, wln_ref[...], preferred_element_type=jnp.float32)
            + jnp.dot(xs, wrp_ref[...], preferred_element_type=jnp.float32)
            + jnp.dot(xs, wrn_ref[...], preferred_element_type=jnp.float32)
        ) * (1.0 / N) + blp_ref[...] + bln_ref[...]
        out_ref[...] = jnp.broadcast_to(r, (8, 128))


_tc_combine = pl.pallas_call(
    _tc_body,
    grid=(NSTEPS,),
    in_specs=[
        pl.BlockSpec((BN, 8), lambda k: (k, 0)),
        pl.BlockSpec((BN, D), lambda k: (k, 0)),
        pl.BlockSpec((D, D), lambda k: (0, 0)),
        pl.BlockSpec((D, D), lambda k: (0, 0)),
        pl.BlockSpec((D, D), lambda k: (0, 0)),
        pl.BlockSpec((D, D), lambda k: (0, 0)),
        pl.BlockSpec((1, D), lambda k: (0, 0)),
        pl.BlockSpec((1, D), lambda k: (0, 0)),
    ],
    out_specs=pl.BlockSpec((8, D), lambda k: (0, 0)),
    out_shape=jax.ShapeDtypeStruct((8, D), jnp.float32),
    scratch_shapes=[pltpu.VMEM((8, D), jnp.float32)],
)


@jax.jit
def kernel(x, edge_index_pos, edge_index_neg,
           Wl_pos, bl_pos, Wr_pos, Wl_neg, bl_neg, Wr_neg):
    if edge_index_pos.dtype != jnp.int32:
        edge_index_pos = edge_index_pos.astype(jnp.int32)
        edge_index_neg = edge_index_neg.astype(jnp.int32)
    srcp = edge_index_pos[0]
    dstp = edge_index_pos[1]
    srcn = edge_index_neg[0]
    dstn = edge_index_neg[1]

    ones_vals = jnp.ones((EPW,), jnp.float32)
    zeros_np = jnp.zeros((NPAD,), jnp.float32)

    s2 = _sc_segment_weights(dstp, srcp, dstn, srcn, ones_vals, zeros_np)

    vt = jnp.concatenate(
        [s2[:, :N].T, jnp.ones((N, 1), jnp.float32), jnp.zeros((N, 5), jnp.float32)],
        axis=1,
    )
    out8 = _tc_combine(
        vt, x, Wl_pos.T, Wl_neg.T, Wr_pos.T, Wr_neg.T,
        bl_pos.reshape(1, D), bl_neg.reshape(1, D),
    )
    return out8[0:1]


# speedup vs baseline: 1.6788x; 1.3931x over previous
import functools

import jax
import jax.numpy as jnp
from jax import lax
from jax.experimental import pallas as pl
from jax.experimental.pallas import tpu as pltpu
from jax.experimental.pallas import tpu_sc as plsc

N = 10000
E = 320000
D = 128
NPAD = 10240
NC = 2
NS = 16
EPW = E // NS


BE = E
ESTEPS = 1


def _detile_body(p_ref, n_ref, sp_ref, dp_ref, sn_ref, dn_ref):
    sp_ref[...] = p_ref[0]
    dp_ref[...] = p_ref[1]
    sn_ref[...] = n_ref[0]
    dn_ref[...] = n_ref[1]


_tc_detile = pl.pallas_call(
    _detile_body,
    grid=(ESTEPS,),
    in_specs=[
        pl.BlockSpec((2, BE), lambda k: (0, k)),
        pl.BlockSpec((2, BE), lambda k: (0, k)),
    ],
    out_specs=[
        pl.BlockSpec((BE,), lambda k: (k,)),
        pl.BlockSpec((BE,), lambda k: (k,)),
        pl.BlockSpec((BE,), lambda k: (k,)),
        pl.BlockSpec((BE,), lambda k: (k,)),
    ],
    out_shape=[jax.ShapeDtypeStruct((E,), jnp.int32) for _ in range(4)],
)


def _sc_mesh():
    return plsc.VectorSubcoreMesh(
        core_axis_name="c", subcore_axis_name="s", num_cores=NC, num_subcores=NS
    )


@functools.partial(
    pl.kernel,
    out_type=jax.ShapeDtypeStruct((NC, NPAD), jnp.float32),
    mesh=_sc_mesh(),
    scratch_types=[
        pltpu.VMEM((EPW,), jnp.int32),
        pltpu.VMEM((EPW,), jnp.int32),
        pltpu.VMEM((EPW,), jnp.float32),
        pltpu.VMEM((EPW,), jnp.float32),
        pltpu.VMEM_SHARED((NPAD,), jnp.float32),
        pltpu.VMEM_SHARED((NPAD,), jnp.float32),
    ],
)
def _sc_segment_weights(
    dstp_hbm, srcp_hbm, dstn_hbm, srcn_hbm, ones_hbm, zeros_hbm,
    out_hbm,
    dst_v, src_v, ones_v, w_v,
    cnt_sh, s_sh,
):
    c = lax.axis_index("c")
    s = lax.axis_index("s")

    base = s * EPW

    @pl.when(c == 0)
    def _():
        pltpu.sync_copy(dstp_hbm.at[pl.ds(base, EPW)], dst_v)
        pltpu.sync_copy(srcp_hbm.at[pl.ds(base, EPW)], src_v)

    @pl.when(c == 1)
    def _():
        pltpu.sync_copy(dstn_hbm.at[pl.ds(base, EPW)], dst_v)
        pltpu.sync_copy(srcn_hbm.at[pl.ds(base, EPW)], src_v)

    pltpu.sync_copy(ones_hbm, ones_v)

    @pl.when(s == 0)
    def _():
        pltpu.sync_copy(zeros_hbm, cnt_sh)
        pltpu.sync_copy(zeros_hbm, s_sh)

    plsc.subcore_barrier()

    pltpu.sync_copy(ones_v, cnt_sh.at[dst_v], add=True)

    plsc.subcore_barrier()

    pltpu.sync_copy(cnt_sh.at[dst_v], w_v)

    def _recip(i, carry):
        for t in range(10):
            sl = pl.ds(i * 160 + t * 16, 16)
            w_v[sl] = 1.0 / w_v[sl]
        return carry

    lax.fori_loop(0, EPW // 160, _recip, 0)
    pltpu.sync_copy(w_v, s_sh.at[src_v], add=True)

    plsc.subcore_barrier()

    @pl.when(s == 0)
    def _():
        pltpu.sync_copy(s_sh, out_hbm.at[c])


def _tc_body(s2_ref, x_ref, wlp_ref, wln_ref, wrp_ref, wrn_ref,
             blp_ref, bln_ref, out_ref):
    v3 = jnp.concatenate(
        [s2_ref[...][:, :N], jnp.ones((1, N), jnp.float32)], axis=0
    )
    y = lax.dot_general(
        v3, x_ref[...], (((1,), (0,)), ((), ())),
        preferred_element_type=jnp.float32,
    )
    sp = y[0:1]
    sn = y[1:2]
    xs = y[2:3]

    def _dot_t(a, w_ref):
        return lax.dot_general(
            a, w_ref[...], (((1,), (1,)), ((), ())),
            preferred_element_type=jnp.float32,
        )

    out_ref[...] = (
        _dot_t(sp, wlp_ref) + _dot_t(sn, wln_ref)
        + _dot_t(xs, wrp_ref) + _dot_t(xs, wrn_ref)
    ) * (1.0 / N) + blp_ref[...] + bln_ref[...]


_tc_combine = pl.pallas_call(
    _tc_body,
    grid=(1,),
    in_specs=[
        pl.BlockSpec((2, NPAD), lambda k: (0, 0)),
        pl.BlockSpec((N, D), lambda k: (0, 0)),
        pl.BlockSpec((D, D), lambda k: (0, 0)),
        pl.BlockSpec((D, D), lambda k: (0, 0)),
        pl.BlockSpec((D, D), lambda k: (0, 0)),
        pl.BlockSpec((D, D), lambda k: (0, 0)),
        pl.BlockSpec((1, D), lambda k: (0, 0)),
        pl.BlockSpec((1, D), lambda k: (0, 0)),
    ],
    out_specs=pl.BlockSpec((1, D), lambda k: (0, 0)),
    out_shape=jax.ShapeDtypeStruct((1, D), jnp.float32),
)


@jax.jit
def kernel(x, edge_index_pos, edge_index_neg,
           Wl_pos, bl_pos, Wr_pos, Wl_neg, bl_neg, Wr_neg):
    if edge_index_pos.dtype != jnp.int32:
        edge_index_pos = edge_index_pos.astype(jnp.int32)
        edge_index_neg = edge_index_neg.astype(jnp.int32)
    srcp, dstp, srcn, dstn = _tc_detile(edge_index_pos, edge_index_neg)

    ones_vals = jnp.ones((EPW,), jnp.float32)
    zeros_np = jnp.zeros((NPAD,), jnp.float32)

    s2 = _sc_segment_weights(dstp, srcp, dstn, srcn, ones_vals, zeros_np)

    return _tc_combine(
        s2, x, Wl_pos, Wl_neg, Wr_pos, Wr_neg,
        bl_pos.reshape(1, D), bl_neg.reshape(1, D),
    )
